# baseline (speedup 1.0000x reference)
import functools

import jax
import jax.numpy as jnp
from jax import lax
from jax.experimental import pallas as pl
from jax.experimental.pallas import tpu as pltpu
from jax.experimental.pallas import tpu_sc as plsc

VOCAB = 1000000
EMBED = 64
BATCH = 16384

NC = 2
NS = 16
NW = NC * NS
B_PER_W = BATCH // NW
CHUNK = 128
NCHUNK = B_PER_W // CHUNK


def _sc_scores(target_r, context_r, in_embed, out_embed):
    @functools.partial(
        pl.kernel,
        mesh=plsc.VectorSubcoreMesh(core_axis_name="c", subcore_axis_name="s"),
        out_type=jax.ShapeDtypeStruct((BATCH,), jnp.float32),
        scratch_types=[
            pltpu.VMEM((NCHUNK, CHUNK), jnp.int32),
            pltpu.VMEM((NCHUNK, CHUNK), jnp.int32),
            pltpu.VMEM((B_PER_W, EMBED), jnp.float32),
            pltpu.VMEM((B_PER_W, EMBED), jnp.float32),
            pltpu.VMEM((B_PER_W,), jnp.float32),
            pltpu.SemaphoreType.DMA,
        ],
        compiler_params=pltpu.CompilerParams(
            needs_layout_passes=False, use_tc_tiling_on_sc=False),
    )
    def body(tgt_hbm, ctx_hbm, ine_hbm, oute_hbm, out_hbm,
             idx_t, idx_c, rows_t, rows_c, score, sem):
        wid = lax.axis_index("s") * NC + lax.axis_index("c")

        pltpu.sync_copy(tgt_hbm.at[wid], idx_t)
        pltpu.sync_copy(ctx_hbm.at[wid], idx_c)

        copies = []
        for j in range(NCHUNK):
            copies.append(pltpu.async_copy(
                ine_hbm.at[idx_t.at[j]],
                rows_t.at[pl.ds(j * CHUNK, CHUNK)], sem))
            copies.append(pltpu.async_copy(
                oute_hbm.at[idx_c.at[j]],
                rows_c.at[pl.ds(j * CHUNK, CHUNK)], sem))
        for c in copies:
            c.wait()

        lanes = lax.iota(jnp.int32, 16)

        def outer(i, _):
            rvec = i * 16 + lanes

            def inner(c, acc):
                cvec = jnp.full((16,), c, dtype=jnp.int32)
                t = plsc.load_gather(rows_t, [rvec, cvec])
                u = plsc.load_gather(rows_c, [rvec, cvec])
                return acc + t * u

            acc = lax.fori_loop(0, EMBED, inner,
                                jnp.zeros((16,), jnp.float32))
            score[pl.ds(i * 16, 16)] = acc
            return 0

        lax.fori_loop(0, B_PER_W // 16, outer, 0)

        pltpu.sync_copy(score, out_hbm.at[pl.ds(wid * B_PER_W, B_PER_W)])

    return body(target_r, context_r, in_embed, out_embed)


def _tc_log_softmax(s_ref, o_ref):
    s = s_ref[...]
    m = jnp.max(s)
    lse = jnp.log(jnp.sum(jnp.exp(s - m))) + m
    o_ref[...] = s - lse


def kernel(target, context, in_embed, out_embed):
    target_r = target.astype(jnp.int32).reshape(NW, NCHUNK, CHUNK)
    context_r = context.astype(jnp.int32).reshape(NW, NCHUNK, CHUNK)
    scores = _sc_scores(target_r, context_r, in_embed, out_embed)
    log_probs = pl.pallas_call(
        _tc_log_softmax,
        out_shape=jax.ShapeDtypeStruct((128, 128), jnp.float32),
    )(scores.reshape(128, 128))
    return log_probs.reshape(-1)
